# baseline (speedup 1.0000x reference)
import jax
import jax.numpy as jnp
from jax import lax
from jax.experimental import pallas as pl
from jax.experimental.pallas import tpu as pltpu
from jax.experimental.pallas import tpu_sc as plsc

DIMS = (119, 4, 12, 12, 10, 6, 6, 2, 2)
D = 128
N = 100000
NC = 9
NW = 32
CHUNK = 256
NCHUNKS = 12
SPW = CHUNK * NCHUNKS
TAIL_START = SPW * NW
TAIL_BLOCKS = (N - TAIL_START) // 16
TBPW = TAIL_BLOCKS // NW
TEXTRA = TAIL_BLOCKS - TBPW * NW

RAW_BASES = (0, 119, 123, 135, 147, 157, 163, 169, 171)
RAW_ROWS = 173
G0, G1, G2, G3 = 0, 119, 215, 335
TABLE_ROWS = 407


def _body(xc0, xc1, xc2, xc3, xc4, xc5, xc6, xc7, xc8,
          emb0, emb1, emb2, emb3, emb4, emb5, emb6, emb7, emb8,
          out_hbm, raw_v, tbl_i, x_v0, x_v1, bs_v0, bs_v1,
          out_v0, out_v1, sem_x0, sem_x1, sem_o0, sem_o1):
    xcols = (xc0, xc1, xc2, xc3, xc4, xc5, xc6, xc7, xc8)
    x_bufs = (x_v0, x_v1)
    bs_bufs = (bs_v0, bs_v1)
    out_bufs = (out_v0, out_v1)
    sem_x = (sem_x0, sem_x1)
    sem_o = (sem_o0, sem_o1)

    embs = (emb0, emb1, emb2, emb3, emb4, emb5, emb6, emb7, emb8)
    for i, e in enumerate(embs):
        pltpu.sync_copy(e, raw_v.at[pl.ds(RAW_BASES[i] * D, DIMS[i] * D)])

    def pack_store(row_out, j, ev, od):
        we = lax.shift_right_logical(plsc.bitcast(ev, jnp.int32) + 0x8000, 16)
        wo = (plsc.bitcast(od, jnp.int32) + 0x8000) & jnp.int32(-65536)
        tbl_i[pl.ds(row_out * (D // 2) + j * 16, 16)] = we | wo

    def build1(base_out, ra, na):
        def row(r, carry):
            for j in range(4):
                pack_store(base_out + r, j,
                           raw_v[pl.ds((ra + r) * D + j * 32, 16)],
                           raw_v[pl.ds((ra + r) * D + j * 32 + 16, 16)])
            return carry
        lax.fori_loop(0, na, row, 0)

    def build2(base_out, ra, na, rb, nb):
        def row(r, carry):
            a = r // nb
            b = r % nb
            for j in range(4):
                ev = (raw_v[pl.ds((ra + a) * D + j * 32, 16)]
                      + raw_v[pl.ds((rb + b) * D + j * 32, 16)])
                od = (raw_v[pl.ds((ra + a) * D + j * 32 + 16, 16)]
                      + raw_v[pl.ds((rb + b) * D + j * 32 + 16, 16)])
                pack_store(base_out + r, j, ev, od)
            return carry
        lax.fori_loop(0, na * nb, row, 0)

    def build3(base_out, ra, na, rb, nb, rc, nc):
        def row(r, carry):
            a = r // nb
            b = r % nb
            for j in range(4):
                se = (raw_v[pl.ds((ra + a) * D + j * 32, 16)]
                      + raw_v[pl.ds((rb + b) * D + j * 32, 16)])
                so = (raw_v[pl.ds((ra + a) * D + j * 32 + 16, 16)]
                      + raw_v[pl.ds((rb + b) * D + j * 32 + 16, 16)])
                for c in range(nc):
                    ev = se + raw_v[pl.ds((rc + c) * D + j * 32, 16)]
                    od = so + raw_v[pl.ds((rc + c) * D + j * 32 + 16, 16)]
                    pack_store(base_out + r * nc + c, j, ev, od)
            return carry
        lax.fori_loop(0, na * nb, row, 0)

    build1(G0, RAW_BASES[0], 119)
    build3(G1, RAW_BASES[1], 4, RAW_BASES[2], 12, RAW_BASES[8], 2)
    build2(G2, RAW_BASES[3], 12, RAW_BASES[4], 10)
    build3(G3, RAW_BASES[5], 6, RAW_BASES[6], 6, RAW_BASES[7], 2)

    lanes = lax.iota(jnp.int32, 16)

    wid = lax.axis_index("s") * 2 + lax.axis_index("c")
    start_sample = wid * SPW

    lanes4 = lanes * 4
    W = D // 2

    def prep_block16(s0, x_v, bases_v):
        xv = [x_v[pl.ds(i * CHUNK + s0, 16)] for i in range(NC)]
        b0v = (xv[0] + G0) * W
        b1v = (xv[1] * 24 + xv[2] * 2 + xv[8] + G1) * W
        b2v = (xv[3] * 10 + xv[4] + G2) * W
        b3v = (xv[5] * 12 + xv[6] * 2 + xv[7] + G3) * W
        ad = lanes4 + s0 * 4
        plsc.store_scatter(bases_v, [ad], b0v)
        plsc.store_scatter(bases_v, [ad + 1], b1v)
        plsc.store_scatter(bases_v, [ad + 2], b2v)
        plsc.store_scatter(bases_v, [ad + 3], b3v)

    def compute_quad(q, bases_v, out_v):
        bvec = bases_v[pl.ds(q * 16, 16)]
        for u in range(4):
            s = q * 4 + u
            b0 = bvec[u * 4 + 0]
            b1 = bvec[u * 4 + 1]
            b2 = bvec[u * 4 + 2]
            b3 = bvec[u * 4 + 3]
            for j in range(4):
                o = j * 16
                w0 = tbl_i[pl.ds(b0 + o, 16)]
                w1 = tbl_i[pl.ds(b1 + o, 16)]
                w2 = tbl_i[pl.ds(b2 + o, 16)]
                w3 = tbl_i[pl.ds(b3 + o, 16)]
                a = (plsc.bitcast(w0, jnp.bfloat16)
                     + plsc.bitcast(w1, jnp.bfloat16))
                bsum = (plsc.bitcast(w2, jnp.bfloat16)
                        + plsc.bitcast(w3, jnp.bfloat16))
                wi = plsc.bitcast(a + bsum, jnp.int32)
                out_v[s, pl.ds(j * 32, 16)] = plsc.bitcast(
                    lax.shift_left(wi, 16), jnp.float32)
                out_v[s, pl.ds(j * 32 + 16, 16)] = plsc.bitcast(
                    wi & jnp.int32(-65536), jnp.float32)

    def start_x(c, b):
        row0 = start_sample + c * CHUNK
        for i in range(NC):
            pltpu.async_copy(xcols[i].at[pl.ds(row0, CHUNK)],
                             x_bufs[b].at[pl.ds(i * CHUNK, CHUNK)], sem_x[b])

    def wait_x(b):
        pltpu.make_async_copy(xc0.at[pl.ds(0, NC * CHUNK)],
                              x_bufs[b].at[pl.ds(0, NC * CHUNK)],
                              sem_x[b]).wait()

    def start_o(c, b):
        row0 = start_sample + c * CHUNK
        pltpu.async_copy(out_bufs[b],
                         out_hbm.at[pl.ds(row0, CHUNK)], sem_o[b])

    def wait_o(b):
        pltpu.make_async_copy(out_bufs[b],
                              out_hbm.at[pl.ds(0, CHUNK)],
                              sem_o[b]).wait()

    start_x(0, 0)
    start_x(1, 1)

    def ring_body(i, carry):
        for b in range(2):
            c = i * 2 + b
            wait_x(b)

            @pl.when(c >= 2)
            def _():
                wait_o(b)

            for blk in range(CHUNK // 16):
                prep_block16(blk * 16, x_bufs[b], bs_bufs[b])

            @plsc.parallel_loop(0, CHUNK // 4, unroll=1)
            def _(q):
                compute_quad(q, bs_bufs[b], out_bufs[b])

            start_o(c, b)

            @pl.when(c + 2 < NCHUNKS)
            def _():
                start_x(c + 2, b)
        return carry

    lax.fori_loop(0, NCHUNKS // 2, ring_body, 0)
    wait_o(0)
    wait_o(1)

    tail_first = wid * TBPW + jnp.minimum(wid, TEXTRA)
    ntail = TBPW + jnp.where(wid < TEXTRA, 1, 0)

    def tail_block(t, carry):
        row0 = TAIL_START + (tail_first + t) * 16
        for i in range(NC):
            pltpu.sync_copy(xcols[i].at[pl.ds(row0, 16)],
                            x_v0.at[pl.ds(i * CHUNK, 16)])
        prep_block16(0, x_v0, bs_v0)

        @plsc.parallel_loop(0, 4, unroll=1)
        def _(q):
            compute_quad(q, bs_v0, out_v0)

        pltpu.sync_copy(out_v0.at[pl.ds(0, 16)],
                        out_hbm.at[pl.ds(row0, 16)])
        return carry

    lax.fori_loop(0, ntail, tail_block, 0)


@jax.jit
def kernel(x, emb0, emb1, emb2, emb3, emb4, emb5, emb6, emb7, emb8):
    mesh = plsc.VectorSubcoreMesh(core_axis_name="c", subcore_axis_name="s")
    run = pl.kernel(
        _body,
        out_type=jax.ShapeDtypeStruct((N, D), jnp.float32),
        mesh=mesh,
        compiler_params=pltpu.CompilerParams(needs_layout_passes=False),
        scratch_types=[
            pltpu.VMEM((RAW_ROWS * D,), jnp.float32),
            pltpu.VMEM((TABLE_ROWS * (D // 2),), jnp.int32),
            pltpu.VMEM((NC * CHUNK,), jnp.int32),
            pltpu.VMEM((NC * CHUNK,), jnp.int32),
            pltpu.VMEM((4 * CHUNK + 16,), jnp.int32),
            pltpu.VMEM((4 * CHUNK + 16,), jnp.int32),
            pltpu.VMEM((CHUNK, D), jnp.float32),
            pltpu.VMEM((CHUNK, D), jnp.float32),
            pltpu.SemaphoreType.DMA,
            pltpu.SemaphoreType.DMA,
            pltpu.SemaphoreType.DMA,
            pltpu.SemaphoreType.DMA,
        ],
    )
    cols = [x[:, i] for i in range(NC)]
    embs = [e.reshape(-1) for e in (emb0, emb1, emb2, emb3, emb4, emb5,
                                    emb6, emb7, emb8)]
    return run(*cols, *embs)


# speedup vs baseline: 1.0609x; 1.0609x over previous
import jax
import jax.numpy as jnp
from jax import lax
from jax.experimental import pallas as pl
from jax.experimental.pallas import tpu as pltpu
from jax.experimental.pallas import tpu_sc as plsc

DIMS = (119, 4, 12, 12, 10, 6, 6, 2, 2)
D = 128
N = 100000
NC = 9
NW = 32
CHUNK = 256
NCHUNKS = 12
SPW = CHUNK * NCHUNKS
TAIL_START = SPW * NW
TAIL_BLOCKS = (N - TAIL_START) // 16
TBPW = TAIL_BLOCKS // NW
TEXTRA = TAIL_BLOCKS - TBPW * NW

RAW_BASES = (0, 119, 123, 135, 147, 157, 163, 169, 171)
RAW_ROWS = 173
G0, G1, G2, G3 = 0, 119, 215, 335
TABLE_ROWS = 407


def _body(xc0, xc1, xc2, xc3, xc4, xc5, xc6, xc7, xc8,
          emb0, emb1, emb2, emb3, emb4, emb5, emb6, emb7, emb8,
          out_hbm, raw_v, tbl_i, x_v0, x_v1, bs_v0, bs_v1,
          out_v0, out_v1, sem_x0, sem_x1, sem_o0, sem_o1):
    xcols = (xc0, xc1, xc2, xc3, xc4, xc5, xc6, xc7, xc8)
    x_bufs = (x_v0, x_v1)
    bs_bufs = (bs_v0, bs_v1)
    out_bufs = (out_v0, out_v1)
    sem_x = (sem_x0, sem_x1)
    sem_o = (sem_o0, sem_o1)

    embs = (emb0, emb1, emb2, emb3, emb4, emb5, emb6, emb7, emb8)
    for i, e in enumerate(embs):
        pltpu.sync_copy(e, raw_v.at[pl.ds(RAW_BASES[i] * D, DIMS[i] * D)])

    def pack_store(row_out, j, ev, od):
        we = lax.shift_right_logical(plsc.bitcast(ev, jnp.int32) + 0x8000, 16)
        wo = (plsc.bitcast(od, jnp.int32) + 0x8000) & jnp.int32(-65536)
        tbl_i[pl.ds(row_out * (D // 2) + j * 16, 16)] = we | wo

    def build1(base_out, ra, na):
        def row(r, carry):
            for j in range(4):
                pack_store(base_out + r, j,
                           raw_v[pl.ds((ra + r) * D + j * 32, 16)],
                           raw_v[pl.ds((ra + r) * D + j * 32 + 16, 16)])
            return carry
        lax.fori_loop(0, na, row, 0)

    def build2(base_out, ra, na, rb, nb):
        def row(r, carry):
            a = r // nb
            b = r % nb
            for j in range(4):
                ev = (raw_v[pl.ds((ra + a) * D + j * 32, 16)]
                      + raw_v[pl.ds((rb + b) * D + j * 32, 16)])
                od = (raw_v[pl.ds((ra + a) * D + j * 32 + 16, 16)]
                      + raw_v[pl.ds((rb + b) * D + j * 32 + 16, 16)])
                pack_store(base_out + r, j, ev, od)
            return carry
        lax.fori_loop(0, na * nb, row, 0)

    def build3(base_out, ra, na, rb, nb, rc, nc):
        def row(r, carry):
            a = r // nb
            b = r % nb
            for j in range(4):
                se = (raw_v[pl.ds((ra + a) * D + j * 32, 16)]
                      + raw_v[pl.ds((rb + b) * D + j * 32, 16)])
                so = (raw_v[pl.ds((ra + a) * D + j * 32 + 16, 16)]
                      + raw_v[pl.ds((rb + b) * D + j * 32 + 16, 16)])
                for c in range(nc):
                    ev = se + raw_v[pl.ds((rc + c) * D + j * 32, 16)]
                    od = so + raw_v[pl.ds((rc + c) * D + j * 32 + 16, 16)]
                    pack_store(base_out + r * nc + c, j, ev, od)
            return carry
        lax.fori_loop(0, na * nb, row, 0)

    build1(G0, RAW_BASES[0], 119)
    build3(G1, RAW_BASES[1], 4, RAW_BASES[2], 12, RAW_BASES[8], 2)
    build2(G2, RAW_BASES[3], 12, RAW_BASES[4], 10)
    build3(G3, RAW_BASES[5], 6, RAW_BASES[6], 6, RAW_BASES[7], 2)

    lanes = lax.iota(jnp.int32, 16)

    wid = lax.axis_index("s") * 2 + lax.axis_index("c")
    start_sample = wid * SPW

    lanes4 = lanes * 4
    W = D // 2

    def prep_block16(s0, x_v, bases_v):
        xv = [x_v[pl.ds(i * CHUNK + s0, 16)] for i in range(NC)]
        b0v = (xv[0] + G0) * W
        b1v = (xv[1] * 24 + xv[2] * 2 + xv[8] + G1) * W
        b2v = (xv[3] * 10 + xv[4] + G2) * W
        b3v = (xv[5] * 12 + xv[6] * 2 + xv[7] + G3) * W
        ad = lanes4 + s0 * 4
        plsc.store_scatter(bases_v, [ad], b0v)
        plsc.store_scatter(bases_v, [ad + 1], b1v)
        plsc.store_scatter(bases_v, [ad + 2], b2v)
        plsc.store_scatter(bases_v, [ad + 3], b3v)

    def compute_sample(s, bases_v, out_v):
        bvec = bases_v[pl.ds(s * 4, 16)]
        b0 = bvec[0]
        b1 = bvec[1]
        b2 = bvec[2]
        b3 = bvec[3]
        for j in range(4):
            o = j * 16
            w0 = tbl_i[pl.ds(b0 + o, 16)]
            w1 = tbl_i[pl.ds(b1 + o, 16)]
            w2 = tbl_i[pl.ds(b2 + o, 16)]
            w3 = tbl_i[pl.ds(b3 + o, 16)]
            a = plsc.bitcast(w0, jnp.bfloat16) + plsc.bitcast(w1, jnp.bfloat16)
            bsum = plsc.bitcast(w2, jnp.bfloat16) + plsc.bitcast(w3, jnp.bfloat16)
            wi = plsc.bitcast(a + bsum, jnp.int32)
            out_v[s, pl.ds(j * 32, 16)] = plsc.bitcast(
                lax.shift_left(wi, 16), jnp.float32)
            out_v[s, pl.ds(j * 32 + 16, 16)] = plsc.bitcast(
                wi & jnp.int32(-65536), jnp.float32)

    def start_x(c, b):
        row0 = start_sample + c * CHUNK
        for i in range(NC):
            pltpu.async_copy(xcols[i].at[pl.ds(row0, CHUNK)],
                             x_bufs[b].at[pl.ds(i * CHUNK, CHUNK)], sem_x[b])

    def wait_x(b):
        pltpu.make_async_copy(xc0.at[pl.ds(0, NC * CHUNK)],
                              x_bufs[b].at[pl.ds(0, NC * CHUNK)],
                              sem_x[b]).wait()

    def start_o(c, b):
        row0 = start_sample + c * CHUNK
        pltpu.async_copy(out_bufs[b],
                         out_hbm.at[pl.ds(row0, CHUNK)], sem_o[b])

    def wait_o(b):
        pltpu.make_async_copy(out_bufs[b],
                              out_hbm.at[pl.ds(0, CHUNK)],
                              sem_o[b]).wait()

    start_x(0, 0)
    start_x(1, 1)

    def ring_body(i, carry):
        for b in range(2):
            c = i * 2 + b
            wait_x(b)

            @pl.when(c >= 2)
            def _():
                wait_o(b)

            for blk in range(CHUNK // 16):
                prep_block16(blk * 16, x_bufs[b], bs_bufs[b])

            @plsc.parallel_loop(0, CHUNK, unroll=4)
            def _(s):
                compute_sample(s, bs_bufs[b], out_bufs[b])

            start_o(c, b)

            @pl.when(c + 2 < NCHUNKS)
            def _():
                start_x(c + 2, b)
        return carry

    lax.fori_loop(0, NCHUNKS // 2, ring_body, 0)
    wait_o(0)
    wait_o(1)

    tail_first = wid * TBPW + jnp.minimum(wid, TEXTRA)
    ntail = TBPW + jnp.where(wid < TEXTRA, 1, 0)

    def tail_block(t, carry):
        row0 = TAIL_START + (tail_first + t) * 16
        for i in range(NC):
            pltpu.sync_copy(xcols[i].at[pl.ds(row0, 16)],
                            x_v0.at[pl.ds(i * CHUNK, 16)])
        prep_block16(0, x_v0, bs_v0)

        @plsc.parallel_loop(0, 16, unroll=4)
        def _(s):
            compute_sample(s, bs_v0, out_v0)

        pltpu.sync_copy(out_v0.at[pl.ds(0, 16)],
                        out_hbm.at[pl.ds(row0, 16)])
        return carry

    lax.fori_loop(0, ntail, tail_block, 0)


@jax.jit
def kernel(x, emb0, emb1, emb2, emb3, emb4, emb5, emb6, emb7, emb8):
    mesh = plsc.VectorSubcoreMesh(core_axis_name="c", subcore_axis_name="s")
    run = pl.kernel(
        _body,
        out_type=jax.ShapeDtypeStruct((N, D), jnp.float32),
        mesh=mesh,
        compiler_params=pltpu.CompilerParams(needs_layout_passes=False),
        scratch_types=[
            pltpu.VMEM((RAW_ROWS * D,), jnp.float32),
            pltpu.VMEM((TABLE_ROWS * (D // 2),), jnp.int32),
            pltpu.VMEM((NC * CHUNK,), jnp.int32),
            pltpu.VMEM((NC * CHUNK,), jnp.int32),
            pltpu.VMEM((4 * CHUNK + 16,), jnp.int32),
            pltpu.VMEM((4 * CHUNK + 16,), jnp.int32),
            pltpu.VMEM((CHUNK, D), jnp.float32),
            pltpu.VMEM((CHUNK, D), jnp.float32),
            pltpu.SemaphoreType.DMA,
            pltpu.SemaphoreType.DMA,
            pltpu.SemaphoreType.DMA,
            pltpu.SemaphoreType.DMA,
        ],
    )
    cols = [x[:, i] for i in range(NC)]
    embs = [e.reshape(-1) for e in (emb0, emb1, emb2, emb3, emb4, emb5,
                                    emb6, emb7, emb8)]
    return run(*cols, *embs)


# speedup vs baseline: 1.1133x; 1.0494x over previous
import jax
import jax.numpy as jnp
from jax import lax
from jax.experimental import pallas as pl
from jax.experimental.pallas import tpu as pltpu
from jax.experimental.pallas import tpu_sc as plsc

DIMS = (119, 4, 12, 12, 10, 6, 6, 2, 2)
D = 128
N = 100000
NC = 9
NW = 32
CHUNK = 256
NCHUNKS = 12
SPW = CHUNK * NCHUNKS
TAIL_START = SPW * NW
TAIL_BLOCKS = (N - TAIL_START) // 16
TBPW = TAIL_BLOCKS // NW
TEXTRA = TAIL_BLOCKS - TBPW * NW

RAW_BASES = (0, 119, 123, 135, 147, 157, 163, 169, 171)
RAW_ROWS = 173
G0, G1, G2, G3 = 0, 119, 215, 335
TABLE_ROWS = 407


def _body(xc0, xc1, xc2, xc3, xc4, xc5, xc6, xc7, xc8,
          emb0, emb1, emb2, emb3, emb4, emb5, emb6, emb7, emb8,
          out_hbm, raw_v, tbl_i, x_v0, x_v1, bs_v0, bs_v1,
          out_v0, out_v1, sem_x0, sem_x1, sem_o0, sem_o1):
    xcols = (xc0, xc1, xc2, xc3, xc4, xc5, xc6, xc7, xc8)
    x_bufs = (x_v0, x_v1)
    bs_bufs = (bs_v0, bs_v1)
    out_bufs = (out_v0, out_v1)
    sem_x = (sem_x0, sem_x1)
    sem_o = (sem_o0, sem_o1)

    embs = (emb0, emb1, emb2, emb3, emb4, emb5, emb6, emb7, emb8)
    for i, e in enumerate(embs):
        pltpu.async_copy(e, raw_v.at[pl.ds(RAW_BASES[i] * D, DIMS[i] * D)],
                         sem_o0)

    wid = lax.axis_index("s") * 2 + lax.axis_index("c")
    start_sample = wid * SPW
    for cc in range(2):
        row0_p = start_sample + cc * CHUNK
        for i in range(NC):
            pltpu.async_copy(xcols[i].at[pl.ds(row0_p, CHUNK)],
                             x_bufs[cc].at[pl.ds(i * CHUNK, CHUNK)],
                             sem_x[cc])

    for i, e in enumerate(embs):
        pltpu.make_async_copy(
            e, raw_v.at[pl.ds(RAW_BASES[i] * D, DIMS[i] * D)], sem_o0).wait()

    def pack_store(row_out, j, ev, od):
        we = lax.shift_right_logical(plsc.bitcast(ev, jnp.int32) + 0x8000, 16)
        wo = (plsc.bitcast(od, jnp.int32) + 0x8000) & jnp.int32(-65536)
        tbl_i[pl.ds(row_out * (D // 2) + j * 16, 16)] = we | wo

    def build1(base_out, ra, na):
        def row(r, carry):
            for j in range(4):
                pack_store(base_out + r, j,
                           raw_v[pl.ds((ra + r) * D + j * 32, 16)],
                           raw_v[pl.ds((ra + r) * D + j * 32 + 16, 16)])
            return carry
        lax.fori_loop(0, na, row, 0)

    def build2(base_out, ra, na, rb, nb):
        def row(r, carry):
            a = r // nb
            b = r % nb
            for j in range(4):
                ev = (raw_v[pl.ds((ra + a) * D + j * 32, 16)]
                      + raw_v[pl.ds((rb + b) * D + j * 32, 16)])
                od = (raw_v[pl.ds((ra + a) * D + j * 32 + 16, 16)]
                      + raw_v[pl.ds((rb + b) * D + j * 32 + 16, 16)])
                pack_store(base_out + r, j, ev, od)
            return carry
        lax.fori_loop(0, na * nb, row, 0)

    def build3(base_out, ra, na, rb, nb, rc, nc):
        def row(r, carry):
            a = r // nb
            b = r % nb
            for j in range(4):
                se = (raw_v[pl.ds((ra + a) * D + j * 32, 16)]
                      + raw_v[pl.ds((rb + b) * D + j * 32, 16)])
                so = (raw_v[pl.ds((ra + a) * D + j * 32 + 16, 16)]
                      + raw_v[pl.ds((rb + b) * D + j * 32 + 16, 16)])
                for c in range(nc):
                    ev = se + raw_v[pl.ds((rc + c) * D + j * 32, 16)]
                    od = so + raw_v[pl.ds((rc + c) * D + j * 32 + 16, 16)]
                    pack_store(base_out + r * nc + c, j, ev, od)
            return carry
        lax.fori_loop(0, na * nb, row, 0)

    build1(G0, RAW_BASES[0], 119)
    build3(G1, RAW_BASES[1], 4, RAW_BASES[2], 12, RAW_BASES[8], 2)
    build2(G2, RAW_BASES[3], 12, RAW_BASES[4], 10)
    build3(G3, RAW_BASES[5], 6, RAW_BASES[6], 6, RAW_BASES[7], 2)

    lanes = lax.iota(jnp.int32, 16)

    lanes4 = lanes * 4
    W = D // 2

    def prep_block16(s0, x_v, bases_v):
        xv = [x_v[pl.ds(i * CHUNK + s0, 16)] for i in range(NC)]
        b0v = (xv[0] + G0) * W
        b1v = (xv[1] * 24 + xv[2] * 2 + xv[8] + G1) * W
        b2v = (xv[3] * 10 + xv[4] + G2) * W
        b3v = (xv[5] * 12 + xv[6] * 2 + xv[7] + G3) * W
        ad = lanes4 + s0 * 4
        plsc.store_scatter(bases_v, [ad], b0v)
        plsc.store_scatter(bases_v, [ad + 1], b1v)
        plsc.store_scatter(bases_v, [ad + 2], b2v)
        plsc.store_scatter(bases_v, [ad + 3], b3v)

    def compute_sample(s, bases_v, out_v):
        bvec = bases_v[pl.ds(s * 4, 16)]
        b0 = bvec[0]
        b1 = bvec[1]
        b2 = bvec[2]
        b3 = bvec[3]
        for j in range(4):
            o = j * 16
            w0 = tbl_i[pl.ds(b0 + o, 16)]
            w1 = tbl_i[pl.ds(b1 + o, 16)]
            w2 = tbl_i[pl.ds(b2 + o, 16)]
            w3 = tbl_i[pl.ds(b3 + o, 16)]
            a = plsc.bitcast(w0, jnp.bfloat16) + plsc.bitcast(w1, jnp.bfloat16)
            bsum = plsc.bitcast(w2, jnp.bfloat16) + plsc.bitcast(w3, jnp.bfloat16)
            wi = plsc.bitcast(a + bsum, jnp.int32)
            out_v[s, pl.ds(j * 32, 16)] = plsc.bitcast(
                lax.shift_left(wi, 16), jnp.float32)
            out_v[s, pl.ds(j * 32 + 16, 16)] = plsc.bitcast(
                wi & jnp.int32(-65536), jnp.float32)

    def start_x(c, b):
        row0 = start_sample + c * CHUNK
        for i in range(NC):
            pltpu.async_copy(xcols[i].at[pl.ds(row0, CHUNK)],
                             x_bufs[b].at[pl.ds(i * CHUNK, CHUNK)], sem_x[b])

    def wait_x(b):
        pltpu.make_async_copy(xc0.at[pl.ds(0, NC * CHUNK)],
                              x_bufs[b].at[pl.ds(0, NC * CHUNK)],
                              sem_x[b]).wait()

    def start_o(c, b):
        row0 = start_sample + c * CHUNK
        pltpu.async_copy(out_bufs[b],
                         out_hbm.at[pl.ds(row0, CHUNK)], sem_o[b])

    def wait_o(b):
        pltpu.make_async_copy(out_bufs[b],
                              out_hbm.at[pl.ds(0, CHUNK)],
                              sem_o[b]).wait()

    def ring_body(i, carry):
        for b in range(2):
            c = i * 2 + b
            wait_x(b)

            @pl.when(c >= 2)
            def _():
                wait_o(b)

            for blk in range(CHUNK // 16):
                prep_block16(blk * 16, x_bufs[b], bs_bufs[b])

            @plsc.parallel_loop(0, CHUNK, unroll=4)
            def _(s):
                compute_sample(s, bs_bufs[b], out_bufs[b])

            start_o(c, b)

            @pl.when(c + 2 < NCHUNKS)
            def _():
                start_x(c + 2, b)
        return carry

    lax.fori_loop(0, NCHUNKS // 2, ring_body, 0)
    wait_o(0)
    wait_o(1)

    tail_first = wid * TBPW + jnp.minimum(wid, TEXTRA)
    ntail = TBPW + jnp.where(wid < TEXTRA, 1, 0)

    def tail_block(t, carry):
        row0 = TAIL_START + (tail_first + t) * 16
        for i in range(NC):
            pltpu.sync_copy(xcols[i].at[pl.ds(row0, 16)],
                            x_v0.at[pl.ds(i * CHUNK, 16)])
        prep_block16(0, x_v0, bs_v0)

        @plsc.parallel_loop(0, 16, unroll=4)
        def _(s):
            compute_sample(s, bs_v0, out_v0)

        pltpu.sync_copy(out_v0.at[pl.ds(0, 16)],
                        out_hbm.at[pl.ds(row0, 16)])
        return carry

    lax.fori_loop(0, ntail, tail_block, 0)


@jax.jit
def kernel(x, emb0, emb1, emb2, emb3, emb4, emb5, emb6, emb7, emb8):
    mesh = plsc.VectorSubcoreMesh(core_axis_name="c", subcore_axis_name="s")
    run = pl.kernel(
        _body,
        out_type=jax.ShapeDtypeStruct((N, D), jnp.float32),
        mesh=mesh,
        compiler_params=pltpu.CompilerParams(needs_layout_passes=False),
        scratch_types=[
            pltpu.VMEM((RAW_ROWS * D,), jnp.float32),
            pltpu.VMEM((TABLE_ROWS * (D // 2),), jnp.int32),
            pltpu.VMEM((NC * CHUNK,), jnp.int32),
            pltpu.VMEM((NC * CHUNK,), jnp.int32),
            pltpu.VMEM((4 * CHUNK + 16,), jnp.int32),
            pltpu.VMEM((4 * CHUNK + 16,), jnp.int32),
            pltpu.VMEM((CHUNK, D), jnp.float32),
            pltpu.VMEM((CHUNK, D), jnp.float32),
            pltpu.SemaphoreType.DMA,
            pltpu.SemaphoreType.DMA,
            pltpu.SemaphoreType.DMA,
            pltpu.SemaphoreType.DMA,
        ],
    )
    cols = [x[:, i] for i in range(NC)]
    embs = [e.reshape(-1) for e in (emb0, emb1, emb2, emb3, emb4, emb5,
                                    emb6, emb7, emb8)]
    return run(*cols, *embs)
